# Initial kernel scaffold; baseline (speedup 1.0000x reference)
import jax
import jax.numpy as jnp
from jax.experimental import pallas as pl


def kernel(inputs, emb_table, special_table):
    raise NotImplementedError("write your pallas kernel here")


# speedup vs baseline: 10.3336x; 10.3336x over previous
import functools

import jax
import jax.numpy as jnp
from jax import lax
from jax.experimental import pallas as pl
from jax.experimental.pallas import tpu as pltpu
from jax.experimental.pallas import tpu_sc as plsc

_NC = 2
_NS = 16
_NW = _NC * _NS
_IPG = 128
_CHUNK = 512


@functools.lru_cache(maxsize=None)
def _make_gather(n_rows: int, vocab: int, d: int):
    assert n_rows % (_NW * _CHUNK) == 0
    b_per_w = n_rows // _NW
    n_chunks = b_per_w // _CHUNK
    n_sub = _CHUNK // _IPG
    idx_rows_per_w = b_per_w // _IPG
    mesh = plsc.VectorSubcoreMesh(core_axis_name="c", subcore_axis_name="s")

    @functools.partial(
        pl.kernel,
        mesh=mesh,
        out_type=jax.ShapeDtypeStruct((n_rows, d), jnp.float32),
        scratch_types=[
            pltpu.VMEM((n_sub, _IPG), jnp.int32),
            pltpu.VMEM((_CHUNK, d), jnp.float32),
            pltpu.SemaphoreType.DMA,
        ],
        compiler_params=pltpu.CompilerParams(use_tc_tiling_on_sc=False),
    )
    def gather_kernel(table_hbm, idx_hbm, out_hbm, idx_v, rows_v, sem):
        wid = lax.axis_index("s") * _NC + lax.axis_index("c")
        row_base = wid * b_per_w
        idx_row_base = wid * idx_rows_per_w

        def body(g, carry):
            pltpu.sync_copy(idx_hbm.at[pl.ds(idx_row_base + g * n_sub, n_sub)],
                            idx_v)
            copies = [
                pltpu.async_copy(table_hbm.at[idx_v.at[j]],
                                 rows_v.at[pl.ds(j * _IPG, _IPG)], sem)
                for j in range(n_sub)
            ]
            for c in copies:
                c.wait()
            pltpu.sync_copy(rows_v, out_hbm.at[pl.ds(row_base + g * _CHUNK,
                                                     _CHUNK)])
            return carry

        lax.fori_loop(0, n_chunks, body, 0)

    return gather_kernel


def _mask_body(x_ref, o_ref):
    o_ref[...] = (x_ref[...] != 0).astype(jnp.int8)


@functools.lru_cache(maxsize=None)
def _make_mask(n_rows: int):
    return pl.pallas_call(
        _mask_body,
        out_shape=jax.ShapeDtypeStruct((n_rows, _IPG), jnp.int8),
    )


def kernel(inputs, emb_table, special_table):
    batch, seq = inputs.shape
    vocab, d = emb_table.shape
    n_special_rows = special_table.shape[0]
    n_valid = vocab - n_special_rows
    combined = emb_table + special_table[0]
    combined = combined.at[n_valid + 1:].add(special_table[1:]
                                             - special_table[0])
    idx2d = inputs.reshape(-1, _IPG).astype(jnp.int32)
    out_flat = _make_gather(batch * seq, vocab, d)(combined, idx2d)
    mask_i8 = _make_mask(idx2d.shape[0])(idx2d)
    return (out_flat.reshape(batch, seq, d),
            mask_i8.reshape(batch, seq).astype(jnp.bool_))
